# baseline (speedup 1.0000x reference)
import functools

import numpy as np
import jax
import jax.numpy as jnp
from jax import lax
from jax.experimental import pallas as pl
from jax.experimental.pallas import tpu as pltpu
from jax.experimental.pallas import tpu_sc as plsc

B = 4096
L = 50
LP1 = L + 1
DIM = 512
VOCAB = 28
MAX_CHARGE = 10

VS = 32
CB = L * VS
TROWS = CB + 16

R = B * LP1
NC = 2
NS = 16
NW = NC * NS
NBR = B // NW
LPAD = 56
ZROW = CB + MAX_CHARGE
NB = 4
NG = NBR // NB


def _positional_encoding_np(length, d_model):
    pos = np.arange(length, dtype=np.float32)[:, None]
    i = np.arange(d_model // 2, dtype=np.float32)[None, :]
    angle = pos / np.power(10000.0, (2.0 * i) / d_model)
    pe = np.zeros((length, d_model), dtype=np.float32)
    pe[:, 0::2] = np.sin(angle)
    pe[:, 1::2] = np.cos(angle)
    return pe


_PE = _positional_encoding_np(L, DIM)


def _prep_body(pe_ref, aa_ref, ch_ref, tok_ref, chg_ref, t3_ref, tc_ref, idx_ref):
    aa = aa_ref[...]
    row = lax.broadcasted_iota(jnp.int32, (VOCAB, DIM), 0)
    aa_z = jnp.where(row == VOCAB - 1, 0.0, aa)
    aa_p = jnp.concatenate(
        [aa_z, jnp.zeros((VS - VOCAB, DIM), jnp.float32)], axis=0)
    pe = pe_ref[...]
    t3_ref[...] = pe[:, None, :] + aa_p[None, :, :]
    ch = ch_ref[...]
    tc_ref[...] = jnp.concatenate(
        [ch, jnp.zeros((TROWS - CB - MAX_CHARGE, DIM), jnp.float32)], axis=0)
    pos_off = lax.broadcasted_iota(jnp.int32, (B, L), 1) * VS
    idx_ref[...] = jnp.concatenate(
        [chg_ref[...] + CB, tok_ref[...] + pos_off,
         jnp.full((B, LPAD - LP1), ZROW, jnp.int32)], axis=1)


def _prep(tokens, charges):
    return pl.pallas_call(
        _prep_body,
        out_shape=[
            jax.ShapeDtypeStruct((L, VS, DIM), jnp.float32),
            jax.ShapeDtypeStruct((TROWS - CB, DIM), jnp.float32),
            jax.ShapeDtypeStruct((B, LPAD), jnp.int32),
        ],
    )


def _gather_body(tab_hbm, idx_hbm, out_hbm, idx_v,
                 b0, b1, b2, b3, g0, g1, g2, g3, s0, s1, s2, s3):
    bufs, gsems, ssems = (b0, b1, b2, b3), (g0, g1, g2, g3), (s0, s1, s2, s3)
    wid = lax.axis_index("s") * NC + lax.axis_index("c")
    bbase = wid * NBR
    pltpu.sync_copy(idx_hbm.at[wid], idx_v)

    def g_start(k, p):
        pltpu.async_copy(tab_hbm.at[idx_v.at[k]], bufs[p], gsems[p])

    def g_wait(p):
        pltpu.make_async_copy(tab_hbm.at[pl.ds(0, LPAD)], bufs[p],
                              gsems[p]).wait()

    def s_start(k, p):
        pltpu.async_copy(bufs[p], out_hbm.at[pl.ds((bbase + k) * LPAD, LPAD)],
                         ssems[p])

    def s_wait(p):
        pltpu.make_async_copy(bufs[p], out_hbm.at[pl.ds(0, LPAD)],
                              ssems[p]).wait()

    g_start(0, 0); g_start(1, 1)
    g_wait(0); s_start(0, 0); g_start(2, 2)
    g_wait(1); s_start(1, 1); g_start(3, 3)
    g_wait(2); s_start(2, 2); s_wait(0); g_start(4, 0)
    g_wait(3); s_start(3, 3); s_wait(1); g_start(5, 1)

    def group(g, carry):
        for s in range(NB):
            k = g * NB + s
            p, p2 = s, (s + 2) % NB
            g_wait(p)
            s_start(k, p)
            s_wait(p2)
            g_start(k + 2, p2)
        return carry

    lax.fori_loop(1, NG - 1, group, 0)

    k0 = (NG - 1) * NB
    g_wait(0); s_start(k0 + 0, 0); s_wait(2); g_start(k0 + 2, 2)
    g_wait(1); s_start(k0 + 1, 1); s_wait(3); g_start(k0 + 3, 3)
    g_wait(2); s_start(k0 + 2, 2); s_wait(0)
    g_wait(3); s_start(k0 + 3, 3); s_wait(1)
    s_wait(2); s_wait(3)


_MESH = plsc.VectorSubcoreMesh(
    core_axis_name="c", subcore_axis_name="s", num_cores=NC, num_subcores=NS)

_sc_gather = functools.partial(
    pl.kernel,
    out_type=jax.ShapeDtypeStruct((B * LPAD, DIM), jnp.float32),
    mesh=_MESH,
    scratch_types=[
        pltpu.VMEM((NBR, LPAD), jnp.int32),
        pltpu.VMEM((LPAD, DIM), jnp.float32),
        pltpu.VMEM((LPAD, DIM), jnp.float32),
        pltpu.VMEM((LPAD, DIM), jnp.float32),
        pltpu.VMEM((LPAD, DIM), jnp.float32),
        pltpu.SemaphoreType.DMA,
        pltpu.SemaphoreType.DMA,
        pltpu.SemaphoreType.DMA,
        pltpu.SemaphoreType.DMA,
        pltpu.SemaphoreType.DMA,
        pltpu.SemaphoreType.DMA,
        pltpu.SemaphoreType.DMA,
        pltpu.SemaphoreType.DMA,
    ],
)(_gather_body)


def _depad_body(pad_ref, out_ref):
    out_ref[...] = pad_ref[:, :LP1, :]


_DB = 32


def _depad(out_pad3):
    return pl.pallas_call(
        _depad_body,
        grid=(B // _DB,),
        in_specs=[pl.BlockSpec((_DB, LPAD, DIM), lambda i: (i, 0, 0))],
        out_specs=pl.BlockSpec((_DB, LP1, DIM), lambda i: (i, 0, 0)),
        out_shape=jax.ShapeDtypeStruct((B, LP1, DIM), jnp.float32),
    )(out_pad3)


def kernel(tokens, charges, aa_table, charge_table):
    tokens = tokens.astype(jnp.int32)
    charges = charges.astype(jnp.int32).reshape(B, 1)
    t3, tc, idx = _prep(tokens, charges)(
        jnp.asarray(_PE), aa_table, charge_table, tokens, charges)
    table = jnp.concatenate([t3.reshape(CB, DIM), tc], axis=0)
    idx3 = idx.reshape(NW, NBR, LPAD)
    out_pad = _sc_gather(table, idx3)
    return _depad(out_pad.reshape(B, LPAD, DIM))


# speedup vs baseline: 2.2955x; 2.2955x over previous
import functools

import numpy as np
import jax
import jax.numpy as jnp
from jax import lax
from jax.experimental import pallas as pl
from jax.experimental.pallas import tpu as pltpu
from jax.experimental.pallas import tpu_sc as plsc

B = 4096
L = 50
LP1 = L + 1
DIM = 512
VOCAB = 28
MAX_CHARGE = 10

VS = 32
CB = L * VS
TROWS = CB + 16

NC = 2
NS = 16
NW = NC * NS

S = 4
BS = B // S
RPS = BS * LP1
C = 96
NCH = RPS // NW // C

DB = 16
NBLK = BS // DB


def _positional_encoding_np(length, d_model):
    pos = np.arange(length, dtype=np.float32)[:, None]
    i = np.arange(d_model // 2, dtype=np.float32)[None, :]
    angle = pos / np.power(10000.0, (2.0 * i) / d_model)
    pe = np.zeros((length, d_model), dtype=np.float32)
    pe[:, 0::2] = np.sin(angle)
    pe[:, 1::2] = np.cos(angle)
    return pe


_PE = _positional_encoding_np(L, DIM)


def _prep_body(pe_ref, aa_ref, ch_ref, tok_ref, chg_ref, t3_ref, tc_ref, idx_ref):
    aa = aa_ref[...]
    row = lax.broadcasted_iota(jnp.int32, (VOCAB, DIM), 0)
    aa_z = jnp.where(row == VOCAB - 1, 0.0, aa)
    aa_p = jnp.concatenate(
        [aa_z, jnp.zeros((VS - VOCAB, DIM), jnp.float32)], axis=0)
    pe = pe_ref[...]
    t3_ref[...] = pe[:, None, :] + aa_p[None, :, :]
    ch = ch_ref[...]
    tc_ref[...] = jnp.concatenate(
        [ch, jnp.zeros((TROWS - CB - MAX_CHARGE, DIM), jnp.float32)], axis=0)
    pos_off = lax.broadcasted_iota(jnp.int32, (B, L), 1) * VS
    idx_ref[...] = jnp.concatenate(
        [chg_ref[...] + CB, tok_ref[...] + pos_off], axis=1)


def _prep(tokens, charges):
    return pl.pallas_call(
        _prep_body,
        out_shape=[
            jax.ShapeDtypeStruct((L, VS, DIM), jnp.float32),
            jax.ShapeDtypeStruct((TROWS - CB, DIM), jnp.float32),
            jax.ShapeDtypeStruct((B, LP1), jnp.int32),
        ],
    )


def _gather_body(tab_hbm, idx_hbm, out_hbm, idx_v, b0, b1, g0, g1, s0, s1):
    bufs, gsems, ssems = (b0, b1), (g0, g1), (s0, s1)
    wid = lax.axis_index("s") * NC + lax.axis_index("c")
    cbase = wid * NCH
    pltpu.sync_copy(idx_hbm.at[wid], idx_v)

    def g_start(k, p):
        pltpu.async_copy(tab_hbm.at[idx_v.at[k]], bufs[p], gsems[p])

    def g_wait(p):
        pltpu.make_async_copy(tab_hbm.at[pl.ds(0, C)], bufs[p],
                              gsems[p]).wait()

    def s_start(k, p):
        pltpu.async_copy(bufs[p], out_hbm.at[pl.ds((cbase + k) * C, C)],
                         ssems[p])

    def s_wait(p):
        pltpu.make_async_copy(bufs[p], out_hbm.at[pl.ds(0, C)],
                              ssems[p]).wait()

    g_start(0, 0)
    g_wait(0); s_start(0, 0); g_start(1, 1)

    def group(g, carry):
        for a in range(2):
            k = 2 * g + 1 + a
            p = 1 - a
            p1 = 1 - p
            g_wait(p)
            s_start(k, p)
            s_wait(p1)
            g_start(k + 1, p1)
        return carry

    lax.fori_loop(0, (NCH - 3) // 2, group, 0)

    g_wait(1); s_start(NCH - 2, 1); s_wait(0); g_start(NCH - 1, 0)
    g_wait(0); s_start(NCH - 1, 0); s_wait(1)
    s_wait(0)


_MESH = plsc.VectorSubcoreMesh(
    core_axis_name="c", subcore_axis_name="s", num_cores=NC, num_subcores=NS)

_sc_gather = functools.partial(
    pl.kernel,
    out_type=jax.ShapeDtypeStruct((RPS, DIM), jnp.float32),
    mesh=_MESH,
    scratch_types=[
        pltpu.VMEM((NCH, C), jnp.int32),
        pltpu.VMEM((C, DIM), jnp.float32),
        pltpu.VMEM((C, DIM), jnp.float32),
        pltpu.SemaphoreType.DMA,
        pltpu.SemaphoreType.DMA,
        pltpu.SemaphoreType.DMA,
        pltpu.SemaphoreType.DMA,
    ],
)(_gather_body)


def _retile_first_body(in_ref, out_ref):
    out_ref[...] = in_ref[...].reshape(DB, LP1, DIM)


def _retile_body(prev_ref, in_ref, out_ref):
    del prev_ref
    out_ref[...] = in_ref[...].reshape(DB, LP1, DIM)


def _retile(s, flat, prev):
    base = s * NBLK
    out_shape = jax.ShapeDtypeStruct((B, LP1, DIM), jnp.float32)
    in_spec = pl.BlockSpec((DB * LP1, DIM), lambda i: (i, 0))
    out_spec = pl.BlockSpec((DB, LP1, DIM), lambda i: (base + i, 0, 0))
    if prev is None:
        return pl.pallas_call(
            _retile_first_body,
            grid=(NBLK,),
            in_specs=[in_spec],
            out_specs=out_spec,
            out_shape=out_shape,
        )(flat)
    return pl.pallas_call(
        _retile_body,
        grid=(NBLK,),
        in_specs=[pl.BlockSpec(memory_space=pl.ANY), in_spec],
        out_specs=out_spec,
        out_shape=out_shape,
        input_output_aliases={0: 0},
    )(prev, flat)


def kernel(tokens, charges, aa_table, charge_table):
    tokens = tokens.astype(jnp.int32)
    charges = charges.astype(jnp.int32).reshape(B, 1)
    t3, tc, idx = _prep(tokens, charges)(
        jnp.asarray(_PE), aa_table, charge_table, tokens, charges)
    table = jnp.concatenate([t3.reshape(CB, DIM), tc], axis=0)
    idx4 = idx.reshape(S, NW, NCH, C)
    out = None
    for s in range(S):
        flat = _sc_gather(table, idx4[s])
        out = _retile(s, flat, out)
    return out
